# baseline (speedup 1.0000x reference)
import functools
import jax
import jax.numpy as jnp
from jax import lax
from jax.experimental import pallas as pl
from jax.experimental.pallas import tpu as pltpu
from jax.experimental.pallas import tpu_sc as plsc

D = 1024
SEQ = 2048
NC, NS = 2, 16
NW = NC * NS
SEQ_PER_W = SEQ // NW
R = 16
SLICES = D // 16
NBLK = SEQ_PER_W // R

_mesh = plsc.VectorSubcoreMesh(core_axis_name="c", subcore_axis_name="s")


@functools.partial(
    pl.kernel,
    mesh=_mesh,
    out_type=jax.ShapeDtypeStruct((SEQ, D), jnp.float32),
    scratch_types=[
        pltpu.VMEM((R, D), jnp.float32),
        pltpu.VMEM((R, D), jnp.float32),
        pltpu.VMEM((R, D), jnp.float32),
        pltpu.SemaphoreType.DMA,
        pltpu.SemaphoreType.DMA,
        pltpu.SemaphoreType.DMA,
        pltpu.SemaphoreType.DMA,
    ],
)
def _sc_add1(x_hbm, emb_hbm, out_hbm, xb0, xb1, ebuf, si0, si1, so0, so1):
    wid = lax.axis_index("s") * NC + lax.axis_index("c")
    s0 = wid * SEQ_PER_W
    xbufs = (xb0, xb1)
    sin = (si0, si1)
    sout = (so0, so1)

    def row_slice(k):
        return pl.ds(s0 + k * R, R)

    def add_block(xbuf):
        def body(i, _):
            sl = pl.ds(i * 16, 16)
            for r in range(R):
                xbuf[r, sl] = xbuf[r, sl] + ebuf[r, sl]
            return 0

        lax.fori_loop(0, SLICES, body, 0)

    pltpu.async_copy(x_hbm.at[row_slice(0), :], xbufs[0], sin[0])
    out_handles = [None, None]
    for k in range(NBLK):
        cur = k % 2
        pltpu.sync_copy(emb_hbm.at[row_slice(k), :], ebuf)
        nxt = k + 1
        if nxt < NBLK:
            nbuf = nxt % 2
            if out_handles[nbuf] is not None:
                out_handles[nbuf].wait()
            pltpu.async_copy(x_hbm.at[row_slice(nxt), :], xbufs[nbuf], sin[nbuf])
        pltpu.make_async_copy(x_hbm.at[row_slice(k), :], xbufs[cur], sin[cur]).wait()
        add_block(xbufs[cur])
        out_handles[cur] = pltpu.make_async_copy(
            xbufs[cur], out_hbm.at[row_slice(k), :], sout[cur]
        )
        out_handles[cur].start()
    for h in out_handles:
        if h is not None:
            h.wait()


def _tc_kernel(x_ref, emb_ref, o_ref):
    o_ref[...] = x_ref[...] + emb_ref[...]


def _tc_add(x, emb):
    batch, seq, d = x.shape
    return pl.pallas_call(
        _tc_kernel,
        grid=(seq // SEQ, batch),
        in_specs=[
            pl.BlockSpec((1, SEQ, d), lambda s, b: (b, s, 0)),
            pl.BlockSpec((SEQ, d), lambda s, b: (s, 0)),
        ],
        out_specs=pl.BlockSpec((1, SEQ, d), lambda s, b: (b, s, 0)),
        out_shape=jax.ShapeDtypeStruct((batch, seq, d), x.dtype),
        compiler_params=pltpu.CompilerParams(
            dimension_semantics=("parallel", "parallel"),
        ),
    )(x, emb)


def kernel(x, emb):
    out_sc = _sc_add1(x[3], emb)
    out_tc = _tc_add(x[:3], emb)
    return jnp.concatenate([out_tc, out_sc[None]], axis=0)
